# baseline (speedup 1.0000x reference)
import jax
import jax.numpy as jnp
from jax import lax
from jax.experimental import pallas as pl
from jax.experimental.pallas import tpu as pltpu
from jax.experimental.pallas import tpu_sc as plsc

N = 10000
E = 320000
D = 128
NSUB = 16
NTILES = 32
EPT = E // NTILES
CH = 80
NCH = EPT // CH
N_ACC = 10112
RPT = N_ACC // NSUB
R_BLK = 1000
CW = 16

_MESH = plsc.VectorSubcoreMesh(core_axis_name="c", subcore_axis_name="s")


NB = 3
_TAIL = (NCH - 2 * NB) // NB * NB + NB


def _agg_body(table, srcs, dsts, zeros, out, src_v, *rest):
  idx_bufs = rest[0:NB]
  rows_bufs = rest[NB:2 * NB]
  in_sems = rest[2 * NB:3 * NB]
  scat_sems = rest[3 * NB:4 * NB]
  acc = rest[4 * NB]
  c = lax.axis_index("c")
  s = lax.axis_index("s")
  wid = c * NSUB + s
  r0 = s * RPT
  pltpu.sync_copy(zeros.at[pl.ds(r0, RPT)], acc.at[pl.ds(r0, RPT)])
  pltpu.sync_copy(srcs.at[wid, 0], src_v)
  plsc.subcore_barrier()

  def start(j, b, first=False):
    if not first:
      pltpu.make_async_copy(rows_bufs[b], acc.at[idx_bufs[b].at[0]],
                            scat_sems[b]).wait()
    pltpu.async_copy(dsts.at[wid, j], idx_bufs[b], in_sems[b])
    pltpu.async_copy(table.at[src_v.at[pl.ds(j * CH, CH)]], rows_bufs[b],
                     in_sems[b])

  def finish(j, b):
    pltpu.make_async_copy(dsts.at[wid, j], idx_bufs[b], in_sems[b]).wait()
    pltpu.make_async_copy(table.at[src_v.at[pl.ds(j * CH, CH)]], rows_bufs[b],
                          in_sems[b]).wait()
    pltpu.async_copy(rows_bufs[b], acc.at[idx_bufs[b].at[0]], scat_sems[b],
                     add=True)

  def drain(b):
    pltpu.make_async_copy(rows_bufs[b], acc.at[idx_bufs[b].at[0]],
                          scat_sems[b]).wait()

  for b in range(NB):
    start(b, b, first=True)

  @pl.loop(0, _TAIL, step=NB)
  def _(j):
    for b in range(NB):
      finish(j + b, b)
    for b in range(NB):
      start(j + NB + b, b)

  for j in range(_TAIL, _TAIL + NB):
    finish(j, j % NB)
  for j in range(_TAIL + NB, NCH):
    start(j, j % NB)
  for j in range(_TAIL + NB, NCH):
    finish(j, j % NB)
  for b in range(NB):
    drain(b)

  plsc.subcore_barrier()
  pltpu.sync_copy(acc.at[pl.ds(r0, RPT)], out.at[c, pl.ds(r0, RPT)])


_agg = pl.kernel(
    _agg_body,
    out_type=jax.ShapeDtypeStruct((2, N_ACC, D), jnp.float32),
    mesh=_MESH,
    compiler_params=pltpu.CompilerParams(use_tc_tiling_on_sc=True),
    scratch_types=(
        [pltpu.VMEM((EPT,), jnp.int32)]
        + [pltpu.VMEM((1, CH), jnp.int32) for _ in range(NB)]
        + [pltpu.VMEM((CH, D), jnp.float32) for _ in range(NB)]
        + [pltpu.SemaphoreType.DMA for _ in range(2 * NB)]
        + [pltpu.VMEM_SHARED((N_ACC, D), jnp.float32)]
    ),
)


def _count_body(dsts, zeros, ones, out, ones_v, *rest):
  idx_bufs = rest[0:NB]
  in_sems = rest[NB:2 * NB]
  scat_sems = rest[2 * NB:3 * NB]
  acc = rest[3 * NB]
  c = lax.axis_index("c")
  s = lax.axis_index("s")
  wid = c * NSUB + s
  r0 = s * RPT
  pltpu.sync_copy(zeros.at[pl.ds(r0, RPT)], acc.at[pl.ds(r0, RPT)])
  pltpu.sync_copy(ones, ones_v)
  plsc.subcore_barrier()

  def start(j, b, first=False):
    if not first:
      pltpu.make_async_copy(ones_v, acc.at[idx_bufs[b].at[0]],
                            scat_sems[b]).wait()
    pltpu.async_copy(dsts.at[wid, j], idx_bufs[b], in_sems[b])

  def finish(j, b):
    pltpu.make_async_copy(dsts.at[wid, j], idx_bufs[b], in_sems[b]).wait()
    pltpu.async_copy(ones_v, acc.at[idx_bufs[b].at[0]], scat_sems[b],
                     add=True)

  def drain(b):
    pltpu.make_async_copy(ones_v, acc.at[idx_bufs[b].at[0]],
                          scat_sems[b]).wait()

  for b in range(NB):
    start(b, b, first=True)

  @pl.loop(0, _TAIL, step=NB)
  def _(j):
    for b in range(NB):
      finish(j + b, b)
    for b in range(NB):
      start(j + NB + b, b)

  for j in range(_TAIL, _TAIL + NB):
    finish(j, j % NB)
  for j in range(_TAIL + NB, NCH):
    start(j, j % NB)
  for j in range(_TAIL + NB, NCH):
    finish(j, j % NB)
  for b in range(NB):
    drain(b)

  plsc.subcore_barrier()
  pltpu.sync_copy(acc.at[pl.ds(r0, RPT)], out.at[c, pl.ds(r0, RPT)])


_count = pl.kernel(
    _count_body,
    out_type=jax.ShapeDtypeStruct((2, N_ACC, CW), jnp.float32),
    mesh=_MESH,
    compiler_params=pltpu.CompilerParams(use_tc_tiling_on_sc=False),
    scratch_types=(
        [pltpu.VMEM((CH, CW), jnp.float32)]
        + [pltpu.VMEM((1, CH), jnp.int32) for _ in range(NB)]
        + [pltpu.SemaphoreType.DMA for _ in range(2 * NB)]
        + [pltpu.VMEM_SHARED((N_ACC, CW), jnp.float32)]
    ),
)

_DN = (((1,), (1,)), ((), ()))


def _make_dense(with_relu):
  def body(p_ref, c_ref, x_ref, wl_ref, wr_ref, b_ref, o_ref):
    sums = p_ref[0] + p_ref[1]
    cnt = c_ref[0, :, :1] + c_ref[1, :, :1]
    mean = sums * (1.0 / jnp.maximum(cnt, 1.0))
    o = (lax.dot_general(mean, wl_ref[...], _DN,
                         preferred_element_type=jnp.float32,
                         precision=lax.Precision.HIGHEST)
         + lax.dot_general(x_ref[...], wr_ref[...], _DN,
                           preferred_element_type=jnp.float32,
                           precision=lax.Precision.HIGHEST)
         + b_ref[...])
    nrm = jnp.sqrt(jnp.sum(o * o, axis=1, keepdims=True))
    o = o / jnp.maximum(nrm, 1e-12)
    if with_relu:
      o = jnp.maximum(o, 0.0)
    o_ref[...] = o

  def run(P, C, x, W_l, W_r, b):
    return pl.pallas_call(
        body,
        grid=(N // R_BLK,),
        in_specs=[
            pl.BlockSpec((2, R_BLK, D), lambda i: (0, i, 0)),
            pl.BlockSpec((2, R_BLK, CW), lambda i: (0, i, 0)),
            pl.BlockSpec((R_BLK, D), lambda i: (i, 0)),
            pl.BlockSpec((D, D), lambda i: (0, 0)),
            pl.BlockSpec((D, D), lambda i: (0, 0)),
            pl.BlockSpec((1, D), lambda i: (0, 0)),
        ],
        out_specs=pl.BlockSpec((R_BLK, D), lambda i: (i, 0)),
        out_shape=jax.ShapeDtypeStruct((N, D), jnp.float32),
    )(P, C, x, W_l, W_r, b)

  return run


_dense_relu = _make_dense(True)
_dense_plain = _make_dense(False)


def kernel(x, edge_index, W_l1, W_r1, b1, W_l2, W_r2, b2):
  src = edge_index[0].astype(jnp.int32).reshape(NTILES, 1, EPT)
  dst = edge_index[1].astype(jnp.int32).reshape(NTILES, NCH, 1, CH)
  zeros = jnp.zeros((N_ACC, D), jnp.float32)
  C = _count(dst, jnp.zeros((N_ACC, CW), jnp.float32),
             jnp.ones((CH, CW), jnp.float32))
  P1 = _agg(x, src, dst, zeros)
  h = _dense_relu(P1, C, x, W_l1, W_r1, b1.reshape(1, D))
  P2 = _agg(h, src, dst, zeros)
  return _dense_plain(P2, C, h, W_l2, W_r2, b2.reshape(1, D))


# speedup vs baseline: 1.1699x; 1.1699x over previous
import jax
import jax.numpy as jnp
from jax import lax
from jax.experimental import pallas as pl
from jax.experimental.pallas import tpu as pltpu
from jax.experimental.pallas import tpu_sc as plsc

N = 10000
E = 320000
D = 128
NSUB = 16
NTILES = 32
EPT = E // NTILES
CH = 80
NCH = EPT // CH
N_ACC = 10112
RPT = N_ACC // NSUB
R_BLK = 1000
CW = 16

_MESH = plsc.VectorSubcoreMesh(core_axis_name="c", subcore_axis_name="s")


NB = 3
_TAIL = (NCH - 2 * NB) // NB * NB + NB


def _agg_body(table, srcs, dsts, zeros, out, src_v, *rest):
  idx_bufs = rest[0:NB]
  rows_bufs = rest[NB:2 * NB]
  in_sems = rest[2 * NB:3 * NB]
  acc = rest[3 * NB]
  c = lax.axis_index("c")
  s = lax.axis_index("s")
  wid = c * NSUB + s
  r0 = s * RPT
  pltpu.sync_copy(zeros.at[pl.ds(r0, RPT)], acc.at[pl.ds(r0, RPT)])
  pltpu.sync_copy(srcs.at[wid, 0], src_v)
  plsc.subcore_barrier()

  def start(j, b):
    pltpu.async_copy(dsts.at[wid, j], idx_bufs[b], in_sems[b])
    pltpu.async_copy(table.at[src_v.at[pl.ds(j * CH, CH)]], rows_bufs[b],
                     in_sems[b])

  def finish(j, b):
    pltpu.make_async_copy(dsts.at[wid, j], idx_bufs[b], in_sems[b]).wait()
    pltpu.make_async_copy(table.at[src_v.at[pl.ds(j * CH, CH)]], rows_bufs[b],
                          in_sems[b]).wait()
    pltpu.sync_copy(rows_bufs[b], acc.at[idx_bufs[b].at[0]], add=True)

  for b in range(NB):
    start(b, b)

  @pl.loop(0, _TAIL, step=NB)
  def _(j):
    for b in range(NB):
      finish(j + b, b)
      start(j + b + NB, b)

  for j in range(_TAIL, NCH):
    finish(j, j % NB)
    if j + NB < NCH:
      start(j + NB, j % NB)

  plsc.subcore_barrier()
  pltpu.sync_copy(acc.at[pl.ds(r0, RPT)], out.at[c, pl.ds(r0, RPT)])


_agg = pl.kernel(
    _agg_body,
    out_type=jax.ShapeDtypeStruct((2, N_ACC, D), jnp.float32),
    mesh=_MESH,
    compiler_params=pltpu.CompilerParams(use_tc_tiling_on_sc=True),
    scratch_types=(
        [pltpu.VMEM((EPT,), jnp.int32)]
        + [pltpu.VMEM((1, CH), jnp.int32) for _ in range(NB)]
        + [pltpu.VMEM((CH, D), jnp.float32) for _ in range(NB)]
        + [pltpu.SemaphoreType.DMA for _ in range(NB)]
        + [pltpu.VMEM_SHARED((N_ACC, D), jnp.float32)]
    ),
)


def _count_body(dsts, zeros, ones, out, ones_v, *rest):
  idx_bufs = rest[0:NB]
  in_sems = rest[NB:2 * NB]
  scat_sems = rest[2 * NB:3 * NB]
  acc = rest[3 * NB]
  c = lax.axis_index("c")
  s = lax.axis_index("s")
  wid = c * NSUB + s
  r0 = s * RPT
  pltpu.sync_copy(zeros.at[pl.ds(r0, RPT)], acc.at[pl.ds(r0, RPT)])
  pltpu.sync_copy(ones, ones_v)
  plsc.subcore_barrier()

  def start(j, b, first=False):
    if not first:
      pltpu.make_async_copy(ones_v, acc.at[idx_bufs[b].at[0]],
                            scat_sems[b]).wait()
    pltpu.async_copy(dsts.at[wid, j], idx_bufs[b], in_sems[b])

  def finish(j, b):
    pltpu.make_async_copy(dsts.at[wid, j], idx_bufs[b], in_sems[b]).wait()
    pltpu.async_copy(ones_v, acc.at[idx_bufs[b].at[0]], scat_sems[b],
                     add=True)

  def drain(b):
    pltpu.make_async_copy(ones_v, acc.at[idx_bufs[b].at[0]],
                          scat_sems[b]).wait()

  for b in range(NB):
    start(b, b, first=True)

  @pl.loop(0, _TAIL, step=NB)
  def _(j):
    for b in range(NB):
      finish(j + b, b)
    for b in range(NB):
      start(j + NB + b, b)

  for j in range(_TAIL, _TAIL + NB):
    finish(j, j % NB)
  for j in range(_TAIL + NB, NCH):
    start(j, j % NB)
  for j in range(_TAIL + NB, NCH):
    finish(j, j % NB)
  for b in range(NB):
    drain(b)

  plsc.subcore_barrier()
  pltpu.sync_copy(acc.at[pl.ds(r0, RPT)], out.at[c, pl.ds(r0, RPT)])


_count = pl.kernel(
    _count_body,
    out_type=jax.ShapeDtypeStruct((2, N_ACC, CW), jnp.float32),
    mesh=_MESH,
    compiler_params=pltpu.CompilerParams(use_tc_tiling_on_sc=False),
    scratch_types=(
        [pltpu.VMEM((CH, CW), jnp.float32)]
        + [pltpu.VMEM((1, CH), jnp.int32) for _ in range(NB)]
        + [pltpu.SemaphoreType.DMA for _ in range(2 * NB)]
        + [pltpu.VMEM_SHARED((N_ACC, CW), jnp.float32)]
    ),
)

_DN = (((1,), (1,)), ((), ()))


def _make_dense(with_relu):
  def body(p_ref, c_ref, x_ref, wl_ref, wr_ref, b_ref, o_ref):
    sums = p_ref[0] + p_ref[1]
    cnt = c_ref[0, :, :1] + c_ref[1, :, :1]
    mean = sums * (1.0 / jnp.maximum(cnt, 1.0))
    o = (lax.dot_general(mean, wl_ref[...], _DN,
                         preferred_element_type=jnp.float32,
                         precision=lax.Precision.HIGHEST)
         + lax.dot_general(x_ref[...], wr_ref[...], _DN,
                           preferred_element_type=jnp.float32,
                           precision=lax.Precision.HIGHEST)
         + b_ref[...])
    nrm = jnp.sqrt(jnp.sum(o * o, axis=1, keepdims=True))
    o = o / jnp.maximum(nrm, 1e-12)
    if with_relu:
      o = jnp.maximum(o, 0.0)
    o_ref[...] = o

  def run(P, C, x, W_l, W_r, b):
    return pl.pallas_call(
        body,
        grid=(N // R_BLK,),
        in_specs=[
            pl.BlockSpec((2, R_BLK, D), lambda i: (0, i, 0)),
            pl.BlockSpec((2, R_BLK, CW), lambda i: (0, i, 0)),
            pl.BlockSpec((R_BLK, D), lambda i: (i, 0)),
            pl.BlockSpec((D, D), lambda i: (0, 0)),
            pl.BlockSpec((D, D), lambda i: (0, 0)),
            pl.BlockSpec((1, D), lambda i: (0, 0)),
        ],
        out_specs=pl.BlockSpec((R_BLK, D), lambda i: (i, 0)),
        out_shape=jax.ShapeDtypeStruct((N, D), jnp.float32),
    )(P, C, x, W_l, W_r, b)

  return run


_dense_relu = _make_dense(True)
_dense_plain = _make_dense(False)


def kernel(x, edge_index, W_l1, W_r1, b1, W_l2, W_r2, b2):
  src = edge_index[0].astype(jnp.int32).reshape(NTILES, 1, EPT)
  dst = edge_index[1].astype(jnp.int32).reshape(NTILES, NCH, 1, CH)
  zeros = jnp.zeros((N_ACC, D), jnp.float32)
  C = _count(dst, jnp.zeros((N_ACC, CW), jnp.float32),
             jnp.ones((CH, CW), jnp.float32))
  P1 = _agg(x, src, dst, zeros)
  h = _dense_relu(P1, C, x, W_l1, W_r1, b1.reshape(1, D))
  P2 = _agg(h, src, dst, zeros)
  return _dense_plain(P2, C, h, W_l2, W_r2, b2.reshape(1, D))


# speedup vs baseline: 1.2492x; 1.0678x over previous
import jax
import jax.numpy as jnp
from jax import lax
from jax.experimental import pallas as pl
from jax.experimental.pallas import tpu as pltpu
from jax.experimental.pallas import tpu_sc as plsc

N = 10000
E = 320000
D = 128
NSUB = 16
NTILES = 32
EPT = E // NTILES
CH = 80
NCH = EPT // CH
N_ACC = 10112
RPT = N_ACC // NSUB
R_BLK = 2000
CW = 16

_MESH = plsc.VectorSubcoreMesh(core_axis_name="c", subcore_axis_name="s")


NB = 3
_TAIL = (NCH - 2 * NB) // NB * NB + NB


def _agg_body(table, srcs, dsts, zeros, out, src_v, *rest):
  idx_bufs = rest[0:NB]
  rows_bufs = rest[NB:2 * NB]
  in_sems = rest[2 * NB:3 * NB]
  acc = rest[3 * NB]
  c = lax.axis_index("c")
  s = lax.axis_index("s")
  wid = c * NSUB + s
  r0 = s * RPT
  pltpu.sync_copy(zeros.at[pl.ds(r0, RPT)], acc.at[pl.ds(r0, RPT)])
  pltpu.sync_copy(srcs.at[wid, 0], src_v)
  plsc.subcore_barrier()

  def start(j, b):
    pltpu.async_copy(dsts.at[wid, j], idx_bufs[b], in_sems[b])
    pltpu.async_copy(table.at[src_v.at[pl.ds(j * CH, CH)]], rows_bufs[b],
                     in_sems[b])

  def finish(j, b):
    pltpu.make_async_copy(dsts.at[wid, j], idx_bufs[b], in_sems[b]).wait()
    pltpu.make_async_copy(table.at[src_v.at[pl.ds(j * CH, CH)]], rows_bufs[b],
                          in_sems[b]).wait()
    pltpu.sync_copy(rows_bufs[b], acc.at[idx_bufs[b].at[0]], add=True)

  for b in range(NB):
    start(b, b)

  @pl.loop(0, _TAIL, step=NB)
  def _(j):
    for b in range(NB):
      finish(j + b, b)
      start(j + b + NB, b)

  for j in range(_TAIL, NCH):
    finish(j, j % NB)
    if j + NB < NCH:
      start(j + NB, j % NB)

  plsc.subcore_barrier()
  pltpu.sync_copy(acc.at[pl.ds(r0, RPT)], out.at[c, pl.ds(r0, RPT)])


_agg = pl.kernel(
    _agg_body,
    out_type=jax.ShapeDtypeStruct((2, N_ACC, D), jnp.float32),
    mesh=_MESH,
    compiler_params=pltpu.CompilerParams(use_tc_tiling_on_sc=True),
    scratch_types=(
        [pltpu.VMEM((EPT,), jnp.int32)]
        + [pltpu.VMEM((1, CH), jnp.int32) for _ in range(NB)]
        + [pltpu.VMEM((CH, D), jnp.float32) for _ in range(NB)]
        + [pltpu.SemaphoreType.DMA for _ in range(NB)]
        + [pltpu.VMEM_SHARED((N_ACC, D), jnp.float32)]
    ),
)


def _count_body(dsts, zeros, ones, out, ones_v, *rest):
  idx_bufs = rest[0:NB]
  in_sems = rest[NB:2 * NB]
  scat_sems = rest[2 * NB:3 * NB]
  acc = rest[3 * NB]
  c = lax.axis_index("c")
  s = lax.axis_index("s")
  wid = c * NSUB + s
  r0 = s * RPT
  pltpu.sync_copy(zeros.at[pl.ds(r0, RPT)], acc.at[pl.ds(r0, RPT)])
  pltpu.sync_copy(ones, ones_v)
  plsc.subcore_barrier()

  def start(j, b, first=False):
    if not first:
      pltpu.make_async_copy(ones_v, acc.at[idx_bufs[b].at[0]],
                            scat_sems[b]).wait()
    pltpu.async_copy(dsts.at[wid, j], idx_bufs[b], in_sems[b])

  def finish(j, b):
    pltpu.make_async_copy(dsts.at[wid, j], idx_bufs[b], in_sems[b]).wait()
    pltpu.async_copy(ones_v, acc.at[idx_bufs[b].at[0]], scat_sems[b],
                     add=True)

  def drain(b):
    pltpu.make_async_copy(ones_v, acc.at[idx_bufs[b].at[0]],
                          scat_sems[b]).wait()

  for b in range(NB):
    start(b, b, first=True)

  @pl.loop(0, _TAIL, step=NB)
  def _(j):
    for b in range(NB):
      finish(j + b, b)
    for b in range(NB):
      start(j + NB + b, b)

  for j in range(_TAIL, _TAIL + NB):
    finish(j, j % NB)
  for j in range(_TAIL + NB, NCH):
    start(j, j % NB)
  for j in range(_TAIL + NB, NCH):
    finish(j, j % NB)
  for b in range(NB):
    drain(b)

  plsc.subcore_barrier()
  pltpu.sync_copy(acc.at[pl.ds(r0, RPT)], out.at[c, pl.ds(r0, RPT)])


_count = pl.kernel(
    _count_body,
    out_type=jax.ShapeDtypeStruct((2, N_ACC, CW), jnp.float32),
    mesh=_MESH,
    compiler_params=pltpu.CompilerParams(use_tc_tiling_on_sc=False),
    scratch_types=(
        [pltpu.VMEM((CH, CW), jnp.float32)]
        + [pltpu.VMEM((1, CH), jnp.int32) for _ in range(NB)]
        + [pltpu.SemaphoreType.DMA for _ in range(2 * NB)]
        + [pltpu.VMEM_SHARED((N_ACC, CW), jnp.float32)]
    ),
)

_DN = (((1,), (1,)), ((), ()))


def _make_dense(with_relu):
  def body(p_ref, c_ref, x_ref, wl_ref, wr_ref, b_ref, o_ref):
    sums = p_ref[0] + p_ref[1]
    cnt = c_ref[0, :, :1] + c_ref[1, :, :1]
    mean = sums * (1.0 / jnp.maximum(cnt, 1.0))
    o = (lax.dot_general(mean, wl_ref[...], _DN,
                         preferred_element_type=jnp.float32)
         + lax.dot_general(x_ref[...], wr_ref[...], _DN,
                           preferred_element_type=jnp.float32)
         + b_ref[...])
    nrm = jnp.sqrt(jnp.sum(o * o, axis=1, keepdims=True))
    o = o / jnp.maximum(nrm, 1e-12)
    if with_relu:
      o = jnp.maximum(o, 0.0)
    o_ref[...] = o

  def run(P, C, x, W_l, W_r, b):
    return pl.pallas_call(
        body,
        grid=(N // R_BLK,),
        in_specs=[
            pl.BlockSpec((2, R_BLK, D), lambda i: (0, i, 0)),
            pl.BlockSpec((2, R_BLK, CW), lambda i: (0, i, 0)),
            pl.BlockSpec((R_BLK, D), lambda i: (i, 0)),
            pl.BlockSpec((D, D), lambda i: (0, 0)),
            pl.BlockSpec((D, D), lambda i: (0, 0)),
            pl.BlockSpec((1, D), lambda i: (0, 0)),
        ],
        out_specs=pl.BlockSpec((R_BLK, D), lambda i: (i, 0)),
        out_shape=jax.ShapeDtypeStruct((N, D), jnp.float32),
    )(P, C, x, W_l, W_r, b)

  return run


_dense_relu = _make_dense(True)
_dense_plain = _make_dense(False)


def kernel(x, edge_index, W_l1, W_r1, b1, W_l2, W_r2, b2):
  src = edge_index[0].astype(jnp.int32).reshape(NTILES, 1, EPT)
  dst = edge_index[1].astype(jnp.int32).reshape(NTILES, NCH, 1, CH)
  zeros = jnp.zeros((N_ACC, D), jnp.float32)
  C = _count(dst, jnp.zeros((N_ACC, CW), jnp.float32),
             jnp.ones((CH, CW), jnp.float32))
  P1 = _agg(x, src, dst, zeros)
  h = _dense_relu(P1, C, x, W_l1, W_r1, b1.reshape(1, D))
  P2 = _agg(h, src, dst, zeros)
  return _dense_plain(P2, C, h, W_l2, W_r2, b2.reshape(1, D))


# speedup vs baseline: 1.2581x; 1.0072x over previous
import jax
import jax.numpy as jnp
from jax import lax
from jax.experimental import pallas as pl
from jax.experimental.pallas import tpu as pltpu
from jax.experimental.pallas import tpu_sc as plsc

N = 10000
E = 320000
D = 128
NSUB = 16
NTILES = 32
EPT = E // NTILES
CH = 80
NCH = EPT // CH
N_ACC = 10112
RPT = N_ACC // NSUB
R_BLK = 2000
CW = 16

_MESH = plsc.VectorSubcoreMesh(core_axis_name="c", subcore_axis_name="s")


NB = 3
_TAIL = (NCH - 2 * NB) // NB * NB + NB


def _agg_body(table, srcs, dsts, zeros, out, src_v, *rest):
  idx_bufs = rest[0:NB]
  rows_bufs = rest[NB:2 * NB]
  in_sems = rest[2 * NB:3 * NB]
  acc = rest[3 * NB]
  c = lax.axis_index("c")
  s = lax.axis_index("s")
  wid = c * NSUB + s
  r0 = s * RPT
  pltpu.async_copy(zeros.at[pl.ds(r0, RPT)], acc.at[pl.ds(r0, RPT)],
                   in_sems[0])
  pltpu.async_copy(srcs.at[wid, 0], src_v, in_sems[1])
  pltpu.make_async_copy(zeros.at[pl.ds(r0, RPT)], acc.at[pl.ds(r0, RPT)],
                        in_sems[0]).wait()
  pltpu.make_async_copy(srcs.at[wid, 0], src_v, in_sems[1]).wait()
  plsc.subcore_barrier()

  def start(j, b):
    pltpu.async_copy(dsts.at[wid, j], idx_bufs[b], in_sems[b])
    pltpu.async_copy(table.at[src_v.at[pl.ds(j * CH, CH)]], rows_bufs[b],
                     in_sems[b])

  def finish(j, b):
    pltpu.make_async_copy(dsts.at[wid, j], idx_bufs[b], in_sems[b]).wait()
    pltpu.make_async_copy(table.at[src_v.at[pl.ds(j * CH, CH)]], rows_bufs[b],
                          in_sems[b]).wait()
    pltpu.sync_copy(rows_bufs[b], acc.at[idx_bufs[b].at[0]], add=True)

  for b in range(NB):
    start(b, b)

  @pl.loop(0, _TAIL, step=NB)
  def _(j):
    for b in range(NB):
      finish(j + b, b)
      start(j + b + NB, b)

  for j in range(_TAIL, NCH):
    finish(j, j % NB)
    if j + NB < NCH:
      start(j + NB, j % NB)

  plsc.subcore_barrier()
  pltpu.sync_copy(acc.at[pl.ds(r0, RPT)], out.at[c, pl.ds(r0, RPT)])


_agg = pl.kernel(
    _agg_body,
    out_type=jax.ShapeDtypeStruct((2, N_ACC, D), jnp.float32),
    mesh=_MESH,
    compiler_params=pltpu.CompilerParams(use_tc_tiling_on_sc=True),
    scratch_types=(
        [pltpu.VMEM((EPT,), jnp.int32)]
        + [pltpu.VMEM((1, CH), jnp.int32) for _ in range(NB)]
        + [pltpu.VMEM((CH, D), jnp.float32) for _ in range(NB)]
        + [pltpu.SemaphoreType.DMA for _ in range(NB)]
        + [pltpu.VMEM_SHARED((N_ACC, D), jnp.float32)]
    ),
)


def _count_body(dsts, zeros, ones, out, ones_v, *rest):
  idx_bufs = rest[0:NB]
  in_sems = rest[NB:2 * NB]
  scat_sems = rest[2 * NB:3 * NB]
  acc = rest[3 * NB]
  c = lax.axis_index("c")
  s = lax.axis_index("s")
  wid = c * NSUB + s
  r0 = s * RPT
  pltpu.sync_copy(zeros.at[pl.ds(r0, RPT)], acc.at[pl.ds(r0, RPT)])
  pltpu.sync_copy(ones, ones_v)
  plsc.subcore_barrier()

  def start(j, b, first=False):
    if not first:
      pltpu.make_async_copy(ones_v, acc.at[idx_bufs[b].at[0]],
                            scat_sems[b]).wait()
    pltpu.async_copy(dsts.at[wid, j], idx_bufs[b], in_sems[b])

  def finish(j, b):
    pltpu.make_async_copy(dsts.at[wid, j], idx_bufs[b], in_sems[b]).wait()
    pltpu.async_copy(ones_v, acc.at[idx_bufs[b].at[0]], scat_sems[b],
                     add=True)

  def drain(b):
    pltpu.make_async_copy(ones_v, acc.at[idx_bufs[b].at[0]],
                          scat_sems[b]).wait()

  for b in range(NB):
    start(b, b, first=True)

  @pl.loop(0, _TAIL, step=NB)
  def _(j):
    for b in range(NB):
      finish(j + b, b)
    for b in range(NB):
      start(j + NB + b, b)

  for j in range(_TAIL, _TAIL + NB):
    finish(j, j % NB)
  for j in range(_TAIL + NB, NCH):
    start(j, j % NB)
  for j in range(_TAIL + NB, NCH):
    finish(j, j % NB)
  for b in range(NB):
    drain(b)

  plsc.subcore_barrier()
  pltpu.sync_copy(acc.at[pl.ds(r0, RPT)], out.at[c, pl.ds(r0, RPT)])


_count = pl.kernel(
    _count_body,
    out_type=jax.ShapeDtypeStruct((2, N_ACC, CW), jnp.float32),
    mesh=_MESH,
    compiler_params=pltpu.CompilerParams(use_tc_tiling_on_sc=False),
    scratch_types=(
        [pltpu.VMEM((CH, CW), jnp.float32)]
        + [pltpu.VMEM((1, CH), jnp.int32) for _ in range(NB)]
        + [pltpu.SemaphoreType.DMA for _ in range(2 * NB)]
        + [pltpu.VMEM_SHARED((N_ACC, CW), jnp.float32)]
    ),
)

_DN = (((1,), (1,)), ((), ()))


def _make_dense(with_relu):
  def body(p_ref, c_ref, x_ref, wl_ref, wr_ref, b_ref, o_ref):
    sums = p_ref[0] + p_ref[1]
    cnt = c_ref[0, :, :1] + c_ref[1, :, :1]
    mean = sums * (1.0 / jnp.maximum(cnt, 1.0))
    o = (lax.dot_general(mean, wl_ref[...], _DN,
                         preferred_element_type=jnp.float32)
         + lax.dot_general(x_ref[...], wr_ref[...], _DN,
                           preferred_element_type=jnp.float32)
         + b_ref[...])
    nrm = jnp.sqrt(jnp.sum(o * o, axis=1, keepdims=True))
    o = o / jnp.maximum(nrm, 1e-12)
    if with_relu:
      o = jnp.maximum(o, 0.0)
    o_ref[...] = o

  def run(P, C, x, W_l, W_r, b):
    return pl.pallas_call(
        body,
        grid=(N // R_BLK,),
        in_specs=[
            pl.BlockSpec((2, R_BLK, D), lambda i: (0, i, 0)),
            pl.BlockSpec((2, R_BLK, CW), lambda i: (0, i, 0)),
            pl.BlockSpec((R_BLK, D), lambda i: (i, 0)),
            pl.BlockSpec((D, D), lambda i: (0, 0)),
            pl.BlockSpec((D, D), lambda i: (0, 0)),
            pl.BlockSpec((1, D), lambda i: (0, 0)),
        ],
        out_specs=pl.BlockSpec((R_BLK, D), lambda i: (i, 0)),
        out_shape=jax.ShapeDtypeStruct((N, D), jnp.float32),
    )(P, C, x, W_l, W_r, b)

  return run


_dense_relu = _make_dense(True)
_dense_plain = _make_dense(False)


def kernel(x, edge_index, W_l1, W_r1, b1, W_l2, W_r2, b2):
  src = edge_index[0].astype(jnp.int32).reshape(NTILES, 1, EPT)
  dst = edge_index[1].astype(jnp.int32).reshape(NTILES, NCH, 1, CH)
  zeros = jnp.zeros((N_ACC, D), jnp.float32)
  C = _count(dst, jnp.zeros((N_ACC, CW), jnp.float32),
             jnp.ones((CH, CW), jnp.float32))
  P1 = _agg(x, src, dst, zeros)
  h = _dense_relu(P1, C, x, W_l1, W_r1, b1.reshape(1, D))
  P2 = _agg(h, src, dst, zeros)
  return _dense_plain(P2, C, h, W_l2, W_r2, b2.reshape(1, D))


# speedup vs baseline: 1.2967x; 1.0307x over previous
import jax
import jax.numpy as jnp
from jax import lax
from jax.experimental import pallas as pl
from jax.experimental.pallas import tpu as pltpu
from jax.experimental.pallas import tpu_sc as plsc

N = 10000
E = 320000
D = 128
NSUB = 16
NTILES = 32
EPT = E // NTILES
CH = 80
NCH = EPT // CH
N_ACC = 10112
RPT = N_ACC // NSUB
R_BLK = 2000
CW = 16

_MESH = plsc.VectorSubcoreMesh(core_axis_name="c", subcore_axis_name="s")


NB = 3
_TAIL = (NCH - 2 * NB) // NB * NB + NB


def _agg_body(table, srcs, dsts, zeros, out, src_v, *rest):
  idx_bufs = rest[0:NB]
  rows_bufs = rest[NB:2 * NB]
  in_sems = rest[2 * NB:3 * NB]
  acc = rest[3 * NB]
  c = lax.axis_index("c")
  s = lax.axis_index("s")
  wid = c * NSUB + s
  r0 = s * RPT
  pltpu.async_copy(zeros.at[pl.ds(r0, RPT)], acc.at[pl.ds(r0, RPT)],
                   in_sems[0])
  pltpu.async_copy(srcs.at[wid, 0], src_v, in_sems[1])
  pltpu.make_async_copy(zeros.at[pl.ds(r0, RPT)], acc.at[pl.ds(r0, RPT)],
                        in_sems[0]).wait()
  pltpu.make_async_copy(srcs.at[wid, 0], src_v, in_sems[1]).wait()
  plsc.subcore_barrier()

  def start(j, b):
    pltpu.async_copy(dsts.at[wid, j], idx_bufs[b], in_sems[b])
    pltpu.async_copy(table.at[src_v.at[pl.ds(j * CH, CH)]], rows_bufs[b],
                     in_sems[b])

  def finish(j, b):
    pltpu.make_async_copy(dsts.at[wid, j], idx_bufs[b], in_sems[b]).wait()
    pltpu.make_async_copy(table.at[src_v.at[pl.ds(j * CH, CH)]], rows_bufs[b],
                          in_sems[b]).wait()
    pltpu.sync_copy(rows_bufs[b], acc.at[idx_bufs[b].at[0]], add=True)

  for b in range(NB):
    start(b, b)

  @pl.loop(0, _TAIL, step=NB)
  def _(j):
    for b in range(NB):
      finish(j + b, b)
      start(j + b + NB, b)

  for j in range(_TAIL, NCH):
    finish(j, j % NB)
    if j + NB < NCH:
      start(j + NB, j % NB)

  plsc.subcore_barrier()
  pltpu.sync_copy(acc.at[pl.ds(r0, RPT)], out.at[c, pl.ds(r0, RPT)])


_agg = pl.kernel(
    _agg_body,
    out_type=jax.ShapeDtypeStruct((2, N_ACC, D), jnp.float32),
    mesh=_MESH,
    compiler_params=pltpu.CompilerParams(use_tc_tiling_on_sc=True),
    scratch_types=(
        [pltpu.VMEM((EPT,), jnp.int32)]
        + [pltpu.VMEM((1, CH), jnp.int32) for _ in range(NB)]
        + [pltpu.VMEM((CH, D), jnp.float32) for _ in range(NB)]
        + [pltpu.SemaphoreType.DMA for _ in range(NB)]
        + [pltpu.VMEM_SHARED((N_ACC, D), jnp.float32)]
    ),
)


NBC = 5
_TAILC = (NCH - 2 * NBC) // NBC * NBC + NBC


def _count_body(dsts, zeros, ones, out, ones_v, *rest):
  idx_bufs = rest[0:NBC]
  in_sems = rest[NBC:2 * NBC]
  scat_sems = rest[2 * NBC:3 * NBC]
  acc = rest[3 * NBC]
  c = lax.axis_index("c")
  s = lax.axis_index("s")
  wid = c * NSUB + s
  r0 = s * RPT
  pltpu.sync_copy(zeros.at[pl.ds(r0, RPT)], acc.at[pl.ds(r0, RPT)])
  pltpu.sync_copy(ones, ones_v)
  plsc.subcore_barrier()

  def start(j, b, first=False):
    if not first:
      pltpu.make_async_copy(ones_v, acc.at[idx_bufs[b].at[0]],
                            scat_sems[b]).wait()
    pltpu.async_copy(dsts.at[wid, j], idx_bufs[b], in_sems[b])

  def finish(j, b):
    pltpu.make_async_copy(dsts.at[wid, j], idx_bufs[b], in_sems[b]).wait()
    pltpu.async_copy(ones_v, acc.at[idx_bufs[b].at[0]], scat_sems[b],
                     add=True)

  def drain(b):
    pltpu.make_async_copy(ones_v, acc.at[idx_bufs[b].at[0]],
                          scat_sems[b]).wait()

  for b in range(NBC):
    start(b, b, first=True)

  @pl.loop(0, _TAILC, step=NBC)
  def _(j):
    for b in range(NBC):
      finish(j + b, b)
    for b in range(NBC):
      start(j + NBC + b, b)

  for j in range(_TAILC, _TAILC + NBC):
    finish(j, j % NBC)
  for j in range(_TAILC + NBC, NCH):
    start(j, j % NBC)
  for j in range(_TAILC + NBC, NCH):
    finish(j, j % NBC)
  for b in range(NBC):
    drain(b)

  plsc.subcore_barrier()
  pltpu.sync_copy(acc.at[pl.ds(r0, RPT)], out.at[c, pl.ds(r0, RPT)])


_count = pl.kernel(
    _count_body,
    out_type=jax.ShapeDtypeStruct((2, N_ACC, CW), jnp.float32),
    mesh=_MESH,
    compiler_params=pltpu.CompilerParams(use_tc_tiling_on_sc=False),
    scratch_types=(
        [pltpu.VMEM((CH, CW), jnp.float32)]
        + [pltpu.VMEM((1, CH), jnp.int32) for _ in range(NBC)]
        + [pltpu.SemaphoreType.DMA for _ in range(2 * NBC)]
        + [pltpu.VMEM_SHARED((N_ACC, CW), jnp.float32)]
    ),
)

_DN = (((1,), (1,)), ((), ()))


def _make_dense(with_relu):
  def body(p_ref, c_ref, x_ref, wl_ref, wr_ref, b_ref, o_ref):
    sums = p_ref[0] + p_ref[1]
    cnt = c_ref[0, :, :1] + c_ref[1, :, :1]
    mean = sums * (1.0 / jnp.maximum(cnt, 1.0))
    o = (lax.dot_general(mean, wl_ref[...], _DN,
                         preferred_element_type=jnp.float32)
         + lax.dot_general(x_ref[...], wr_ref[...], _DN,
                           preferred_element_type=jnp.float32)
         + b_ref[...])
    nrm = jnp.sqrt(jnp.sum(o * o, axis=1, keepdims=True))
    o = o / jnp.maximum(nrm, 1e-12)
    if with_relu:
      o = jnp.maximum(o, 0.0)
    o_ref[...] = o

  def run(P, C, x, W_l, W_r, b):
    return pl.pallas_call(
        body,
        grid=(N // R_BLK,),
        in_specs=[
            pl.BlockSpec((2, R_BLK, D), lambda i: (0, i, 0)),
            pl.BlockSpec((2, R_BLK, CW), lambda i: (0, i, 0)),
            pl.BlockSpec((R_BLK, D), lambda i: (i, 0)),
            pl.BlockSpec((D, D), lambda i: (0, 0)),
            pl.BlockSpec((D, D), lambda i: (0, 0)),
            pl.BlockSpec((1, D), lambda i: (0, 0)),
        ],
        out_specs=pl.BlockSpec((R_BLK, D), lambda i: (i, 0)),
        out_shape=jax.ShapeDtypeStruct((N, D), jnp.float32),
    )(P, C, x, W_l, W_r, b)

  return run


_dense_relu = _make_dense(True)
_dense_plain = _make_dense(False)


def kernel(x, edge_index, W_l1, W_r1, b1, W_l2, W_r2, b2):
  src = edge_index[0].astype(jnp.int32).reshape(NTILES, 1, EPT)
  dst = edge_index[1].astype(jnp.int32).reshape(NTILES, NCH, 1, CH)
  zeros = jnp.zeros((N_ACC, D), jnp.float32)
  C = _count(dst, jnp.zeros((N_ACC, CW), jnp.float32),
             jnp.ones((CH, CW), jnp.float32))
  P1 = _agg(x, src, dst, zeros)
  h = _dense_relu(P1, C, x, W_l1, W_r1, b1.reshape(1, D))
  P2 = _agg(h, src, dst, zeros)
  return _dense_plain(P2, C, h, W_l2, W_r2, b2.reshape(1, D))


# speedup vs baseline: 1.2969x; 1.0002x over previous
import jax
import jax.numpy as jnp
from jax import lax
from jax.experimental import pallas as pl
from jax.experimental.pallas import tpu as pltpu
from jax.experimental.pallas import tpu_sc as plsc

N = 10000
E = 320000
D = 128
NSUB = 16
NTILES = 32
EPT = E // NTILES
CH = 80
NCH = EPT // CH
N_ACC = 10112
RPT = N_ACC // NSUB
R_BLK = 2000
CW = 16

_MESH = plsc.VectorSubcoreMesh(core_axis_name="c", subcore_axis_name="s")


NB = 3
_TAIL = (NCH - 2 * NB) // NB * NB + NB


def _agg_body(table, srcs, dsts, zeros, out, src_v, *rest):
  idx_bufs = rest[0:NB]
  rows_bufs = rest[NB:2 * NB]
  in_sems = rest[2 * NB:3 * NB]
  acc = rest[3 * NB]
  c = lax.axis_index("c")
  s = lax.axis_index("s")
  wid = c * NSUB + s
  r0 = s * RPT
  pltpu.async_copy(zeros.at[pl.ds(r0, RPT)], acc.at[pl.ds(r0, RPT)],
                   in_sems[0])
  pltpu.async_copy(srcs.at[wid, 0], src_v, in_sems[1])
  pltpu.make_async_copy(zeros.at[pl.ds(r0, RPT)], acc.at[pl.ds(r0, RPT)],
                        in_sems[0]).wait()
  pltpu.make_async_copy(srcs.at[wid, 0], src_v, in_sems[1]).wait()
  plsc.subcore_barrier()

  def start(j, b):
    pltpu.async_copy(dsts.at[wid, j], idx_bufs[b], in_sems[b])
    pltpu.async_copy(table.at[src_v.at[pl.ds(j * CH, CH)]], rows_bufs[b],
                     in_sems[b])

  def finish(j, b):
    pltpu.make_async_copy(dsts.at[wid, j], idx_bufs[b], in_sems[b]).wait()
    pltpu.make_async_copy(table.at[src_v.at[pl.ds(j * CH, CH)]], rows_bufs[b],
                          in_sems[b]).wait()
    pltpu.sync_copy(rows_bufs[b], acc.at[idx_bufs[b].at[0]], add=True)

  for b in range(NB):
    start(b, b)

  @pl.loop(0, _TAIL, step=NB)
  def _(j):
    for b in range(NB):
      finish(j + b, b)
      start(j + b + NB, b)

  for j in range(_TAIL, NCH):
    finish(j, j % NB)
    if j + NB < NCH:
      start(j + NB, j % NB)

  plsc.subcore_barrier()
  pltpu.sync_copy(acc.at[pl.ds(r0, RPT)], out.at[c, pl.ds(r0, RPT)])


_agg = pl.kernel(
    _agg_body,
    out_type=jax.ShapeDtypeStruct((2, N_ACC, D), jnp.float32),
    mesh=_MESH,
    compiler_params=pltpu.CompilerParams(use_tc_tiling_on_sc=True),
    scratch_types=(
        [pltpu.VMEM((EPT,), jnp.int32)]
        + [pltpu.VMEM((1, CH), jnp.int32) for _ in range(NB)]
        + [pltpu.VMEM((CH, D), jnp.float32) for _ in range(NB)]
        + [pltpu.SemaphoreType.DMA for _ in range(NB)]
        + [pltpu.VMEM_SHARED((N_ACC, D), jnp.float32)]
    ),
)


NBC = 5
_TAILC = (NCH - 2 * NBC) // NBC * NBC + NBC


def _count_body(dsts, zeros, ones, out, ones_v, *rest):
  idx_bufs = rest[0:NBC]
  in_sems = rest[NBC:2 * NBC]
  acc = rest[2 * NBC]
  c = lax.axis_index("c")
  s = lax.axis_index("s")
  wid = c * NSUB + s
  r0 = s * RPT
  pltpu.sync_copy(zeros.at[pl.ds(r0, RPT)], acc.at[pl.ds(r0, RPT)])
  pltpu.sync_copy(ones, ones_v)
  plsc.subcore_barrier()

  def start(j, b):
    pltpu.async_copy(dsts.at[wid, j], idx_bufs[b], in_sems[b])

  def finish(j, b):
    pltpu.make_async_copy(dsts.at[wid, j], idx_bufs[b], in_sems[b]).wait()
    pltpu.sync_copy(ones_v, acc.at[idx_bufs[b].at[0]], add=True)

  for b in range(NBC):
    start(b, b)

  @pl.loop(0, _TAILC, step=NBC)
  def _(j):
    for b in range(NBC):
      finish(j + b, b)
      start(j + b + NBC, b)

  for j in range(_TAILC, NCH):
    finish(j, j % NBC)
    if j + NBC < NCH:
      start(j + NBC, j % NBC)

  plsc.subcore_barrier()
  pltpu.sync_copy(acc.at[pl.ds(r0, RPT)], out.at[c, pl.ds(r0, RPT)])


_count = pl.kernel(
    _count_body,
    out_type=jax.ShapeDtypeStruct((2, N_ACC, CW), jnp.float32),
    mesh=_MESH,
    compiler_params=pltpu.CompilerParams(use_tc_tiling_on_sc=False),
    scratch_types=(
        [pltpu.VMEM((CH, CW), jnp.float32)]
        + [pltpu.VMEM((1, CH), jnp.int32) for _ in range(NBC)]
        + [pltpu.SemaphoreType.DMA for _ in range(NBC)]
        + [pltpu.VMEM_SHARED((N_ACC, CW), jnp.float32)]
    ),
)

_DN = (((1,), (1,)), ((), ()))


def _make_dense(with_relu):
  def body(p_ref, c_ref, x_ref, wl_ref, wr_ref, b_ref, o_ref):
    sums = p_ref[0] + p_ref[1]
    cnt = c_ref[0, :, :1] + c_ref[1, :, :1]
    mean = sums * (1.0 / jnp.maximum(cnt, 1.0))
    o = (lax.dot_general(mean, wl_ref[...], _DN,
                         preferred_element_type=jnp.float32)
         + lax.dot_general(x_ref[...], wr_ref[...], _DN,
                           preferred_element_type=jnp.float32)
         + b_ref[...])
    nrm = jnp.sqrt(jnp.sum(o * o, axis=1, keepdims=True))
    o = o / jnp.maximum(nrm, 1e-12)
    if with_relu:
      o = jnp.maximum(o, 0.0)
    o_ref[...] = o

  def run(P, C, x, W_l, W_r, b):
    return pl.pallas_call(
        body,
        grid=(N // R_BLK,),
        in_specs=[
            pl.BlockSpec((2, R_BLK, D), lambda i: (0, i, 0)),
            pl.BlockSpec((2, R_BLK, CW), lambda i: (0, i, 0)),
            pl.BlockSpec((R_BLK, D), lambda i: (i, 0)),
            pl.BlockSpec((D, D), lambda i: (0, 0)),
            pl.BlockSpec((D, D), lambda i: (0, 0)),
            pl.BlockSpec((1, D), lambda i: (0, 0)),
        ],
        out_specs=pl.BlockSpec((R_BLK, D), lambda i: (i, 0)),
        out_shape=jax.ShapeDtypeStruct((N, D), jnp.float32),
    )(P, C, x, W_l, W_r, b)

  return run


_dense_relu = _make_dense(True)
_dense_plain = _make_dense(False)


def kernel(x, edge_index, W_l1, W_r1, b1, W_l2, W_r2, b2):
  src = edge_index[0].astype(jnp.int32).reshape(NTILES, 1, EPT)
  dst = edge_index[1].astype(jnp.int32).reshape(NTILES, NCH, 1, CH)
  zeros = jnp.zeros((N_ACC, D), jnp.float32)
  C = _count(dst, jnp.zeros((N_ACC, CW), jnp.float32),
             jnp.ones((CH, CW), jnp.float32))
  P1 = _agg(x, src, dst, zeros)
  h = _dense_relu(P1, C, x, W_l1, W_r1, b1.reshape(1, D))
  P2 = _agg(h, src, dst, zeros)
  return _dense_plain(P2, C, h, W_l2, W_r2, b2.reshape(1, D))
